# initial kernel scaffold (unmeasured)
import jax
import jax.numpy as jnp
from jax import lax
from jax.experimental import pallas as pl
from jax.experimental.pallas import tpu as pltpu

N_DEV = 16


def kernel(x, w_mat):
    k_total, m_per = x.shape
    assert k_total == N_DEV * m_per
    n = w_mat.shape[1]

    def body(x_ref, w_ref, out_ref, xrow_ref, send_sems, recv_sems):
        my_i = lax.axis_index("i")

        xrow_ref[:, pl.ds(my_i * m_per, m_per)] = x_ref[pl.ds(my_i * m_per, m_per), :]

        sends = []
        for off in range(1, N_DEV):
            dst = (my_i + off) % N_DEV
            rdma = pltpu.make_async_remote_copy(
                src_ref=x_ref.at[pl.ds(dst * m_per, m_per), :],
                dst_ref=xrow_ref.at[:, pl.ds(my_i * m_per, m_per)],
                send_sem=send_sems.at[off - 1],
                recv_sem=recv_sems.at[off - 1],
                device_id=(dst,),
                device_id_type=pl.DeviceIdType.MESH,
            )
            rdma.start()
            sends.append(rdma)

        for off in range(1, N_DEV):
            src = (my_i - off) % N_DEV
            recv = pltpu.make_async_remote_copy(
                src_ref=x_ref.at[pl.ds(src * m_per, m_per), :],
                dst_ref=xrow_ref.at[:, pl.ds(src * m_per, m_per)],
                send_sem=send_sems.at[off - 1],
                recv_sem=recv_sems.at[off - 1],
                device_id=(src,),
                device_id_type=pl.DeviceIdType.MESH,
            )
            recv.wait_recv()

        for rdma in sends:
            rdma.wait_send()

        acc = jnp.dot(
            xrow_ref[:, :], w_ref[:, :], preferred_element_type=jnp.float32
        )
        out_ref[:, :] = jnp.maximum(acc, 0.0)

    return pl.pallas_call(
        body,
        out_shape=jax.ShapeDtypeStruct((m_per, n), jnp.float32),
        in_specs=[
            pl.BlockSpec(memory_space=pltpu.VMEM),
            pl.BlockSpec(memory_space=pltpu.VMEM),
        ],
        out_specs=pl.BlockSpec(memory_space=pltpu.VMEM),
        scratch_shapes=[
            pltpu.VMEM((m_per, k_total), x.dtype),
            pltpu.SemaphoreType.DMA((N_DEV - 1,)),
            pltpu.SemaphoreType.DMA((N_DEV - 1,)),
        ],
        compiler_params=pltpu.CompilerParams(collective_id=0),
    )(x, w_mat)


# baseline (device time: 20258 ns/iter reference)
import jax
import jax.numpy as jnp
from jax import lax
from jax.experimental import pallas as pl
from jax.experimental.pallas import tpu as pltpu

N_DEV = 16


def kernel(x, w_mat):
    k_total, m_per = x.shape
    assert k_total == N_DEV * m_per
    n = w_mat.shape[1]

    def body(x_ref, w_ref, out_ref, xgat_ref, send_sems, recv_sems):
        my_i = lax.axis_index("i")

        xgat_ref[my_i, :, :] = x_ref[pl.ds(my_i * m_per, m_per), :]

        sends = []
        for off in range(1, N_DEV):
            dst = (my_i + off) % N_DEV
            rdma = pltpu.make_async_remote_copy(
                src_ref=x_ref.at[pl.ds(dst * m_per, m_per), :],
                dst_ref=xgat_ref.at[my_i],
                send_sem=send_sems.at[off - 1],
                recv_sem=recv_sems.at[off - 1],
                device_id=(dst,),
                device_id_type=pl.DeviceIdType.MESH,
            )
            rdma.start()
            sends.append(rdma)

        for off in range(1, N_DEV):
            src = (my_i - off) % N_DEV
            recv = pltpu.make_async_remote_copy(
                src_ref=x_ref.at[pl.ds(src * m_per, m_per), :],
                dst_ref=xgat_ref.at[src],
                send_sem=send_sems.at[off - 1],
                recv_sem=recv_sems.at[off - 1],
                device_id=(src,),
                device_id_type=pl.DeviceIdType.MESH,
            )
            recv.wait_recv()

        for rdma in sends:
            rdma.wait_send()

        acc = jnp.zeros((m_per, n), dtype=jnp.float32)
        for s in range(N_DEV):
            acc += jnp.dot(
                xgat_ref[s, :, :],
                w_ref[s * m_per:(s + 1) * m_per, :],
                preferred_element_type=jnp.float32,
            )
        out_ref[:, :] = jnp.maximum(acc, 0.0)

    return pl.pallas_call(
        body,
        out_shape=jax.ShapeDtypeStruct((m_per, n), jnp.float32),
        in_specs=[
            pl.BlockSpec(memory_space=pltpu.VMEM),
            pl.BlockSpec(memory_space=pltpu.VMEM),
        ],
        out_specs=pl.BlockSpec(memory_space=pltpu.VMEM),
        scratch_shapes=[
            pltpu.VMEM((N_DEV, m_per, m_per), x.dtype),
            pltpu.SemaphoreType.DMA((N_DEV - 1,)),
            pltpu.SemaphoreType.DMA((N_DEV - 1,)),
        ],
    )(x, w_mat)
